# Initial kernel scaffold; baseline (speedup 1.0000x reference)
import jax
import jax.numpy as jnp
from jax.experimental import pallas as pl


def kernel(x, table):
    raise NotImplementedError("write your pallas kernel here")


# speedup vs baseline: 7.1649x; 7.1649x over previous
import functools

import jax
import jax.numpy as jnp
from jax import lax
from jax.experimental import pallas as pl
from jax.experimental.pallas import tpu as pltpu
from jax.experimental.pallas import tpu_sc as plsc

_NUM_CORES = 2
_NUM_SUBCORES = 16
_NUM_WORKERS = _NUM_CORES * _NUM_SUBCORES
_FIRE = 8


@functools.cache
def _build(seq_len: int, table_rows: int, depth: int):
    rows_per_w = seq_len // _NUM_WORKERS
    n_chunks = rows_per_w // _FIRE

    def body(rev_hbm, out_hbm, spmem, sem):
        c = lax.axis_index("c")
        s = lax.axis_index("s")

        @pl.when(s == 0)
        def _stage():
            pltpu.sync_copy(rev_hbm, spmem)

        plsc.subcore_barrier()

        wid = c * _NUM_SUBCORES + s
        base = wid * rows_per_w
        for k in range(n_chunks):
            copies = []
            for r in range(_FIRE):
                i = base + (k * _FIRE + r)
                start = (seq_len - 1) - i
                copies.append(pltpu.make_async_copy(
                    spmem.at[pl.ds(start, seq_len)], out_hbm.at[i], sem))
            for cp in copies:
                cp.start()
            for cp in copies:
                cp.wait()

    mesh = plsc.VectorSubcoreMesh(core_axis_name="c", subcore_axis_name="s")
    return pl.kernel(
        body,
        out_type=jax.ShapeDtypeStruct((seq_len, seq_len, depth), jnp.float32),
        mesh=mesh,
        scratch_types=[
            pltpu.VMEM_SHARED((table_rows, depth), jnp.float32),
            pltpu.SemaphoreType.DMA,
        ],
    )


def kernel(x, table):
    seq_len = x.shape[1]
    rev = table[::-1]
    return _build(seq_len, table.shape[0], table.shape[1])(rev)


# speedup vs baseline: 82.0735x; 11.4550x over previous
import functools

import jax
import jax.numpy as jnp
from jax import lax
from jax.experimental import pallas as pl
from jax.experimental.pallas import tpu as pltpu
from jax.experimental.pallas import tpu_sc as plsc

_NUM_CORES = 2
_NUM_SUBCORES = 16
_NUM_WORKERS = _NUM_CORES * _NUM_SUBCORES


@functools.cache
def _build(seq_len: int, depth: int, src_pitch: int):
    rows_per_w = seq_len // _NUM_WORKERS
    win_pitch = seq_len + rows_per_w
    kt_tiles = depth // 8
    jt_tiles = seq_len // 128
    block = jt_tiles * 8 * 128
    n_blocks = rows_per_w * kt_tiles
    row_words = seq_len * depth

    def body(rf_hbm, out_hbm, win, buf, sem_a, sem_b, stage_sem):
        c = lax.axis_index("c")
        s = lax.axis_index("s")
        wid = c * _NUM_SUBCORES + s
        base = wid * rows_per_w
        wstart = (seq_len - rows_per_w) - base
        stages = [
            pltpu.make_async_copy(
                rf_hbm.at[pl.ds(k * src_pitch + wstart, win_pitch)],
                win.at[pl.ds(k * win_pitch, win_pitch)],
                stage_sem,
            )
            for k in range(depth)
        ]
        for cp in stages:
            cp.start()
        for cp in stages:
            cp.wait()

        def step(rc, carry):
            kt = lax.rem(rc, kt_tiles)
            r = lax.div(rc, kt_tiles)
            local = (rows_per_w - 1) - r
            sbase = kt * (8 * win_pitch) + local
            slot = lax.rem(rc, 2)
            dst = out_hbm.at[pl.ds(base * row_words + rc * block, block)]
            bslot = buf.at[pl.ds(slot * block, block)]

            @pl.when(jnp.logical_and(slot == 0, rc >= 2))
            def _wait_a():
                pltpu.make_async_copy(bslot, dst, sem_a).wait()

            @pl.when(jnp.logical_and(slot == 1, rc >= 2))
            def _wait_b():
                pltpu.make_async_copy(bslot, dst, sem_b).wait()

            dbase = slot * block
            batch = 16
            ngroups = block // 16 // batch

            def _load(v):
                jt, kr, t = v // 64, (v // 8) % 8, v % 8
                soff = sbase + kr * win_pitch + jt * 128 + t * 16
                return win[pl.ds(soff, 16)]

            def _store(v, val):
                buf[pl.ds(dbase + v * 16, 16)] = val

            prev = [_load(u) for u in range(batch)]
            for g in range(1, ngroups):
                cur = []
                for u in range(batch):
                    cur.append(_load(g * batch + u))
                    _store((g - 1) * batch + u, prev[u])
                prev = cur
            for u in range(batch):
                _store((ngroups - 1) * batch + u, prev[u])

            @pl.when(slot == 0)
            def _send_a():
                pltpu.make_async_copy(bslot, dst, sem_a).start()

            @pl.when(slot == 1)
            def _send_b():
                pltpu.make_async_copy(bslot, dst, sem_b).start()

            return carry

        lax.fori_loop(0, n_blocks, step, 0)
        tail = out_hbm.at[pl.ds(base * row_words, block)]
        pltpu.make_async_copy(buf.at[pl.ds(0, block)], tail, sem_a).wait()
        pltpu.make_async_copy(buf.at[pl.ds(0, block)], tail, sem_b).wait()

    mesh = plsc.VectorSubcoreMesh(core_axis_name="c", subcore_axis_name="s")
    return pl.kernel(
        body,
        out_type=jax.ShapeDtypeStruct((seq_len * seq_len * depth,), jnp.float32),
        mesh=mesh,
        scratch_types=[
            pltpu.VMEM((depth * win_pitch,), jnp.float32),
            pltpu.VMEM((2 * block,), jnp.float32),
            pltpu.SemaphoreType.DMA,
            pltpu.SemaphoreType.DMA,
            pltpu.SemaphoreType.DMA,
        ],
    )


def kernel(x, table):
    seq_len = x.shape[1]
    depth = table.shape[1]
    revt = table[::-1].T
    src_pitch = -(-(revt.shape[1] + 1) // 128) * 128
    revt = jnp.pad(revt, ((0, 0), (0, src_pitch - revt.shape[1])))
    rf = revt.reshape(-1)
    out1d = _build(seq_len, depth, src_pitch)(rf)
    b = out1d.reshape(seq_len, depth // 8, seq_len // 128, 8, 128)
    return b.transpose(0, 2, 4, 1, 3).reshape(seq_len, seq_len, depth)


# speedup vs baseline: 82.1983x; 1.0015x over previous
import functools

import jax
import jax.numpy as jnp
from jax import lax
from jax.experimental import pallas as pl
from jax.experimental.pallas import tpu as pltpu
from jax.experimental.pallas import tpu_sc as plsc

_NUM_CORES = 2
_NUM_SUBCORES = 16
_NUM_WORKERS = _NUM_CORES * _NUM_SUBCORES


@functools.cache
def _build(seq_len: int, depth: int, src_pitch: int):
    rows_per_w = seq_len // _NUM_WORKERS
    win_pitch = seq_len + rows_per_w
    kt_tiles = depth // 8
    jt_tiles = seq_len // 128
    block = jt_tiles * 8 * 128
    n_blocks = rows_per_w * kt_tiles
    row_words = seq_len * depth

    def body(rf_hbm, out_hbm, win, buf, sem_a, sem_b, stage_sem):
        c = lax.axis_index("c")
        s = lax.axis_index("s")
        wid = c * _NUM_SUBCORES + s
        base = wid * rows_per_w
        wstart = (seq_len - rows_per_w) - base
        stages = [
            pltpu.make_async_copy(
                rf_hbm.at[pl.ds(k * src_pitch + wstart, win_pitch)],
                win.at[pl.ds(k * win_pitch, win_pitch)],
                stage_sem,
            )
            for k in range(depth)
        ]
        for cp in stages:
            cp.start()
        for cp in stages:
            cp.wait()

        def step(rc, carry):
            kt = lax.rem(rc, kt_tiles)
            r = lax.div(rc, kt_tiles)
            local = (rows_per_w - 1) - r
            sbase = kt * (8 * win_pitch) + local
            slot = lax.rem(rc, 2)
            dst = out_hbm.at[pl.ds(base * row_words + rc * block, block)]
            bslot = buf.at[pl.ds(slot * block, block)]

            @pl.when(jnp.logical_and(slot == 0, rc >= 2))
            def _wait_a():
                pltpu.make_async_copy(bslot, dst, sem_a).wait()

            @pl.when(jnp.logical_and(slot == 1, rc >= 2))
            def _wait_b():
                pltpu.make_async_copy(bslot, dst, sem_b).wait()

            dbase = slot * block
            batch = 8
            ngroups = block // 16 // batch

            def _load(v):
                jt, kr, t = v // 64, (v // 8) % 8, v % 8
                soff = sbase + kr * win_pitch + jt * 128 + t * 16
                return win[pl.ds(soff, 16)]

            def _store(v, val):
                buf[pl.ds(dbase + v * 16, 16)] = val

            prev = [_load(u) for u in range(batch)]
            for g in range(1, ngroups):
                cur = []
                for u in range(batch):
                    cur.append(_load(g * batch + u))
                    _store((g - 1) * batch + u, prev[u])
                prev = cur
            for u in range(batch):
                _store((ngroups - 1) * batch + u, prev[u])

            @pl.when(slot == 0)
            def _send_a():
                pltpu.make_async_copy(bslot, dst, sem_a).start()

            @pl.when(slot == 1)
            def _send_b():
                pltpu.make_async_copy(bslot, dst, sem_b).start()

            return carry

        lax.fori_loop(0, n_blocks, step, 0)
        tail = out_hbm.at[pl.ds(base * row_words, block)]
        pltpu.make_async_copy(buf.at[pl.ds(0, block)], tail, sem_a).wait()
        pltpu.make_async_copy(buf.at[pl.ds(0, block)], tail, sem_b).wait()

    mesh = plsc.VectorSubcoreMesh(core_axis_name="c", subcore_axis_name="s")
    return pl.kernel(
        body,
        out_type=jax.ShapeDtypeStruct((seq_len * seq_len * depth,), jnp.float32),
        mesh=mesh,
        scratch_types=[
            pltpu.VMEM((depth * win_pitch,), jnp.float32),
            pltpu.VMEM((2 * block,), jnp.float32),
            pltpu.SemaphoreType.DMA,
            pltpu.SemaphoreType.DMA,
            pltpu.SemaphoreType.DMA,
        ],
    )


def kernel(x, table):
    seq_len = x.shape[1]
    depth = table.shape[1]
    revt = table[::-1].T
    src_pitch = -(-(revt.shape[1] + 1) // 128) * 128
    revt = jnp.pad(revt, ((0, 0), (0, src_pitch - revt.shape[1])))
    rf = revt.reshape(-1)
    out1d = _build(seq_len, depth, src_pitch)(rf)
    b = out1d.reshape(seq_len, depth // 8, seq_len // 128, 8, 128)
    return b.transpose(0, 2, 4, 1, 3).reshape(seq_len, seq_len, depth)
